# baseline (speedup 1.0000x reference)
import functools

import jax
import jax.numpy as jnp
from jax import lax
from jax.experimental import pallas as pl
from jax.experimental.pallas import tpu as pltpu
from jax.experimental.pallas import tpu_sc as plsc


def _pos_emb_sc(pe, idx_t):
    n = idx_t.shape[0] // 2
    d = pe.shape[1]

    mesh = plsc.VectorSubcoreMesh(core_axis_name="c", subcore_axis_name="s")

    @functools.partial(
        pl.kernel,
        mesh=mesh,
        out_type=jax.ShapeDtypeStruct((n, 2 * d), jnp.float32),
        scratch_types=[
            pltpu.VMEM((n,), jnp.int32),
            pltpu.VMEM((n, d), jnp.float32),
            pltpu.SemaphoreType.DMA,
        ],
    )
    def gather_kernel(pe_hbm, idx_hbm, out_hbm, idx_v, rows_v, sem):
        cid = lax.axis_index("c")
        sid = lax.axis_index("s")

        def do_half(half):
            pltpu.sync_copy(idx_hbm.at[pl.ds(half * n, n)], idx_v)
            pltpu.async_copy(pe_hbm.at[idx_v], rows_v, sem).wait()
            pltpu.sync_copy(rows_v, out_hbm.at[pl.ds(0, n), pl.ds(half * d, d)])

        @pl.when((sid == 0) & (cid == 0))
        def _():
            do_half(0)

        @pl.when((sid == 0) & (cid == 1))
        def _():
            do_half(1)

    return gather_kernel(pe, idx_t)


def _add_body(x_ref, p_ref, o_ref):
    o_ref[...] = x_ref[...] + p_ref[...]


def kernel(x, positions, pe):
    B, C, H, W = x.shape
    idx_t = positions.T.reshape(-1).astype(jnp.int32)
    pos_emb = _pos_emb_sc(pe, idx_t)

    rows_total = B * C
    row_blk = 256
    x3 = x.reshape(rows_total, H, W)

    out = pl.pallas_call(
        _add_body,
        grid=(rows_total // row_blk,),
        in_specs=[
            pl.BlockSpec((row_blk, H, W), lambda i: (i, 0, 0)),
            pl.BlockSpec((H, W), lambda i: (0, 0)),
        ],
        out_specs=pl.BlockSpec((row_blk, H, W), lambda i: (i, 0, 0)),
        out_shape=jax.ShapeDtypeStruct((rows_total, H, W), x.dtype),
        compiler_params=pltpu.CompilerParams(
            dimension_semantics=("arbitrary",),
        ),
    )(x3, pos_emb)

    return out.reshape(B, C, H, W)


# speedup vs baseline: 1.0262x; 1.0262x over previous
import functools

import jax
import jax.numpy as jnp
from jax import lax
from jax.experimental import pallas as pl
from jax.experimental.pallas import tpu as pltpu
from jax.experimental.pallas import tpu_sc as plsc


def _pos_emb_sc(pe, idx_t):
    n = idx_t.shape[0] // 2
    d = pe.shape[1]

    mesh = plsc.VectorSubcoreMesh(core_axis_name="c", subcore_axis_name="s", num_cores=1)

    @functools.partial(
        pl.kernel,
        mesh=mesh,
        out_type=jax.ShapeDtypeStruct((n, 2 * d), jnp.float32),
        scratch_types=[
            pltpu.VMEM((n,), jnp.int32),
            pltpu.VMEM((n, d), jnp.float32),
            pltpu.SemaphoreType.DMA,
        ],
    )
    def gather_kernel(pe_hbm, idx_hbm, out_hbm, idx_v, rows_v, sem):
        cid = lax.axis_index("c")
        sid = lax.axis_index("s")

        def do_half(half):
            pltpu.sync_copy(idx_hbm.at[pl.ds(half * n, n)], idx_v)
            pltpu.async_copy(pe_hbm.at[idx_v], rows_v, sem).wait()
            pltpu.sync_copy(rows_v, out_hbm.at[pl.ds(0, n), pl.ds(half * d, d)])

        del cid
        @pl.when(sid == 0)
        def _():
            do_half(0)

        @pl.when(sid == 1)
        def _():
            do_half(1)

    return gather_kernel(pe, idx_t)


def _add_body(x_ref, p_ref, o_ref):
    o_ref[...] = x_ref[...] + p_ref[...]


def kernel(x, positions, pe):
    B, C, H, W = x.shape
    idx_t = positions.T.reshape(-1).astype(jnp.int32)
    pos_emb = _pos_emb_sc(pe, idx_t)

    rows_total = B * C
    row_blk = 256
    x3 = x.reshape(rows_total, H, W)

    out = pl.pallas_call(
        _add_body,
        grid=(rows_total // row_blk,),
        in_specs=[
            pl.BlockSpec((row_blk, H, W), lambda i: (i, 0, 0)),
            pl.BlockSpec((H, W), lambda i: (0, 0)),
        ],
        out_specs=pl.BlockSpec((row_blk, H, W), lambda i: (i, 0, 0)),
        out_shape=jax.ShapeDtypeStruct((rows_total, H, W), x.dtype),
        compiler_params=pltpu.CompilerParams(
            dimension_semantics=("arbitrary",),
        ),
    )(x3, pos_emb)

    return out.reshape(B, C, H, W)


# speedup vs baseline: 1.0530x; 1.0261x over previous
import functools

import jax
import jax.numpy as jnp
from jax import lax
from jax.experimental import pallas as pl
from jax.experimental.pallas import tpu as pltpu
from jax.experimental.pallas import tpu_sc as plsc


def _pos_emb_sc(pe, idx_t):
    n = idx_t.shape[0] // 2
    d = pe.shape[1]

    mesh = plsc.VectorSubcoreMesh(core_axis_name="c", subcore_axis_name="s", num_cores=1)

    @functools.partial(
        pl.kernel,
        mesh=mesh,
        out_type=jax.ShapeDtypeStruct((n, 2 * d), jnp.float32),
        scratch_types=[
            pltpu.VMEM((n,), jnp.int32),
            pltpu.VMEM((n, d), jnp.float32),
            pltpu.SemaphoreType.DMA,
        ],
    )
    def gather_kernel(pe_hbm, idx_hbm, out_hbm, idx_v, rows_v, sem):
        cid = lax.axis_index("c")
        sid = lax.axis_index("s")

        def do_half(half):
            pltpu.sync_copy(idx_hbm.at[pl.ds(half * n, n)], idx_v)
            pltpu.async_copy(pe_hbm.at[idx_v], rows_v, sem).wait()
            pltpu.sync_copy(rows_v, out_hbm.at[pl.ds(0, n), pl.ds(half * d, d)])

        del cid
        @pl.when(sid == 0)
        def _():
            do_half(0)

        @pl.when(sid == 1)
        def _():
            do_half(1)

    return gather_kernel(pe, idx_t)


def _add_body(x_ref, p_ref, o_ref):
    o_ref[...] = x_ref[...] + p_ref[...]


def kernel(x, positions, pe):
    B, C, H, W = x.shape
    idx_t = positions.T.reshape(-1).astype(jnp.int32)
    pos_emb = _pos_emb_sc(pe, idx_t)

    rows_total = B * C
    row_blk = 512
    x3 = x.reshape(rows_total, H, W)

    out = pl.pallas_call(
        _add_body,
        grid=(rows_total // row_blk,),
        in_specs=[
            pl.BlockSpec((row_blk, H, W), lambda i: (i, 0, 0)),
            pl.BlockSpec((H, W), lambda i: (0, 0)),
        ],
        out_specs=pl.BlockSpec((row_blk, H, W), lambda i: (i, 0, 0)),
        out_shape=jax.ShapeDtypeStruct((rows_total, H, W), x.dtype),
        compiler_params=pltpu.CompilerParams(
            dimension_semantics=("arbitrary",),
        ),
    )(x3, pos_emb)

    return out.reshape(B, C, H, W)


# speedup vs baseline: 1.0538x; 1.0007x over previous
import functools

import jax
import jax.numpy as jnp
from jax import lax
from jax.experimental import pallas as pl
from jax.experimental.pallas import tpu as pltpu
from jax.experimental.pallas import tpu_sc as plsc


def _pos_emb_sc(pe, idx_t):
    n = idx_t.shape[0] // 2
    d = pe.shape[1]

    mesh = plsc.VectorSubcoreMesh(core_axis_name="c", subcore_axis_name="s", num_cores=1)

    @functools.partial(
        pl.kernel,
        mesh=mesh,
        out_type=jax.ShapeDtypeStruct((n, 2 * d), jnp.float32),
        scratch_types=[
            pltpu.VMEM((n,), jnp.int32),
            pltpu.VMEM((n, d), jnp.float32),
            pltpu.SemaphoreType.DMA,
        ],
    )
    def gather_kernel(pe_hbm, idx_hbm, out_hbm, idx_v, rows_v, sem):
        cid = lax.axis_index("c")
        sid = lax.axis_index("s")

        def do_half(half):
            pltpu.sync_copy(idx_hbm.at[pl.ds(half * n, n)], idx_v)
            pltpu.async_copy(pe_hbm.at[idx_v], rows_v, sem).wait()
            pltpu.sync_copy(rows_v, out_hbm.at[pl.ds(0, n), pl.ds(half * d, d)])

        del cid
        @pl.when(sid == 0)
        def _():
            do_half(0)

        @pl.when(sid == 1)
        def _():
            do_half(1)

    return gather_kernel(pe, idx_t)


def _add_body(x_ref, p_ref, o_ref):
    o_ref[...] = x_ref[...] + p_ref[...]


def kernel(x, positions, pe):
    B, C, H, W = x.shape
    idx_t = positions.T.reshape(-1).astype(jnp.int32)
    pos_emb = _pos_emb_sc(pe, idx_t)

    rows_total = B * C
    row_blk = 512
    x3 = x.reshape(rows_total, H, W)

    out = pl.pallas_call(
        _add_body,
        grid=(rows_total // row_blk,),
        in_specs=[
            pl.BlockSpec((row_blk, H, W), lambda i: (i, 0, 0)),
            pl.BlockSpec((H, W), lambda i: (0, 0)),
        ],
        out_specs=pl.BlockSpec((row_blk, H, W), lambda i: (i, 0, 0)),
        out_shape=jax.ShapeDtypeStruct((rows_total, H, W), x.dtype),
        compiler_params=pltpu.CompilerParams(
            dimension_semantics=("parallel",),
        ),
    )(x3, pos_emb)

    return out.reshape(B, C, H, W)
